# baseline (speedup 1.0000x reference)
import functools

import jax
import jax.numpy as jnp
from jax.experimental import pallas as pl
from jax.experimental.pallas import tpu as pltpu
from jax.experimental.pallas import tpu_sc as plsc

N_EMB = 8192
DIM = 32
N_TOK = 8192
TILE = 512
GRID = N_TOK // TILE

_NC = 2
_NS = 16
_NW = _NC * _NS
_BPW = N_TOK // _NW


_PAD = 128

_SCALE = 20.0 * 1.4426950408889634


def _prep_body(cb_ref, cbn_ref, cbnp_ref):
    cb = cb_ref[...]
    n = jnp.sqrt(jnp.sum(cb * cb, axis=1, keepdims=True))
    cbn = cb / jnp.maximum(n, 1e-12)
    cbn_ref[...] = cbn
    cbnp_ref[...] = jnp.concatenate(
        [cbn, jnp.zeros((N_EMB, _PAD - DIM), jnp.float32)], axis=1)


def _vq_body(x_ref, cb_ref, loss_ref, soft_ref, idx_ref, lsum_ref):
    i = pl.program_id(0)

    @pl.when(i == 0)
    def _init():
        lsum_ref[0, 0] = 0.0

    x = x_ref[...]
    xn = x / jnp.maximum(jnp.sqrt(jnp.sum(x * x, axis=1, keepdims=True)), 1e-12)
    cbn = cb_ref[...]
    logits = jax.lax.dot_general(xn, cbn, (((1,), (1,)), ((), ())),
                                 preferred_element_type=jnp.float32)
    d = 2.0 - 2.0 * logits
    m = jnp.max(logits, axis=1, keepdims=True)
    e = jnp.exp2(logits * _SCALE - m * _SCALE)
    r = 1.0 / jnp.sum(e, axis=1, keepdims=True)
    soft_ref[...] = e * r

    idx = jnp.argmin(d, axis=1).astype(jnp.int32)
    idx_ref[0, 0, :] = idx
    lsum_ref[0, 0] += jnp.sum(m)

    @pl.when(i == GRID - 1)
    def _fin():
        loss_ref[...] = jnp.reshape(
            0.25 * (2.0 * N_TOK - 2.0 * lsum_ref[0, 0]) / (N_TOK * DIM),
            (1, 1))


def _sc_body(cbn_hbm, idx_hbm, quant_hbm, counts_hbm, idx_v, rows_v, table_v,
             sem):
    c = jax.lax.axis_index("c")
    s = jax.lax.axis_index("s")
    wid = s * _NC + c
    base = wid * _BPW
    pltpu.sync_copy(idx_hbm.at[pl.ds(base, _BPW)], idx_v)
    pltpu.async_copy(cbn_hbm.at[idx_v], rows_v, sem).wait()
    pltpu.sync_copy(rows_v, quant_hbm.at[pl.ds(base, _BPW)])

    def _zero(j, carry):
        table_v[pl.ds(j * 16, 16)] = jnp.zeros((16,), jnp.float32)
        return carry

    jax.lax.fori_loop(0, N_EMB // 16, _zero, 0)

    def _hist(j, carry):
        idx16 = idx_v[pl.ds(j * 16, 16)]
        plsc.addupdate_scatter(table_v, [idx16], jnp.ones((16,), jnp.float32))
        return carry

    jax.lax.fori_loop(0, _BPW // 16, _hist, 0)
    pltpu.sync_copy(table_v, counts_hbm.at[wid])


_sc_gather_hist = functools.partial(
    pl.kernel,
    mesh=plsc.VectorSubcoreMesh(core_axis_name="c", subcore_axis_name="s"),
    out_type=[
        jax.ShapeDtypeStruct((N_TOK, _PAD), jnp.float32),
        jax.ShapeDtypeStruct((_NW, N_EMB), jnp.float32),
    ],
    scratch_types=[
        pltpu.VMEM((_BPW,), jnp.int32),
        pltpu.VMEM((_BPW, _PAD), jnp.float32),
        pltpu.VMEM((N_EMB,), jnp.float32),
        pltpu.SemaphoreType.DMA,
    ],
    compiler_params=pltpu.CompilerParams(needs_layout_passes=False),
)(_sc_body)


def _perp_body(cnt_ref, perp_ref):
    counts = jnp.sum(cnt_ref[...], axis=0, keepdims=True)
    avg = counts / N_TOK
    perp_ref[...] = jnp.reshape(
        jnp.exp(-jnp.sum(avg * jnp.log(avg + 1e-10))), (1, 1))


def kernel(inputs, codebook):
    flat = inputs.reshape(-1, DIM)
    cbn, cbnp = pl.pallas_call(
        _prep_body,
        out_shape=[
            jax.ShapeDtypeStruct((N_EMB, DIM), jnp.float32),
            jax.ShapeDtypeStruct((N_EMB, _PAD), jnp.float32),
        ],
    )(codebook)
    loss, soft, idx = pl.pallas_call(
        _vq_body,
        grid=(GRID,),
        in_specs=[
            pl.BlockSpec((TILE, DIM), lambda i: (i, 0)),
            pl.BlockSpec((N_EMB, DIM), lambda i: (0, 0)),
        ],
        out_specs=[
            pl.BlockSpec((1, 1), lambda i: (0, 0)),
            pl.BlockSpec((TILE, N_EMB), lambda i: (i, 0)),
            pl.BlockSpec((1, 1, TILE), lambda i: (i, 0, 0)),
        ],
        out_shape=[
            jax.ShapeDtypeStruct((1, 1), jnp.float32),
            jax.ShapeDtypeStruct((N_TOK, N_EMB), jnp.float32),
            jax.ShapeDtypeStruct((GRID, 1, TILE), jnp.int32),
        ],
        scratch_shapes=[
            pltpu.SMEM((1, 1), jnp.float32),
        ],
    )(flat, cbn)
    idx_flat = idx.reshape(-1)
    quant_pad, counts = _sc_gather_hist(cbnp, idx_flat)
    quant = quant_pad[:, :DIM]
    perp = pl.pallas_call(
        _perp_body,
        out_shape=jax.ShapeDtypeStruct((1, 1), jnp.float32),
    )(counts)
    return (loss[0, 0], quant.reshape(inputs.shape), soft, perp[0, 0],
            idx_flat[:, None])


# speedup vs baseline: 1.2233x; 1.2233x over previous
import functools

import jax
import jax.numpy as jnp
from jax.experimental import pallas as pl
from jax.experimental.pallas import tpu as pltpu
from jax.experimental.pallas import tpu_sc as plsc

N_EMB = 8192
DIM = 32
N_TOK = 8192
TILE = 512
GRID = N_TOK // TILE

_NC = 2
_NS = 16
_NW = _NC * _NS
_BPW = N_TOK // _NW


_PAD = 128

_NEGS = -10.0 * 1.4426950408889634


def _prep_body(cb_ref, cbn_ref, cbnp_ref):
    cb = cb_ref[...]
    n = jnp.sqrt(jnp.sum(cb * cb, axis=1, keepdims=True))
    cbn = cb / jnp.maximum(n, 1e-12)
    cbn_ref[...] = cbn
    cbnp_ref[...] = jnp.concatenate(
        [cbn, jnp.zeros((N_EMB, _PAD - DIM), jnp.float32)], axis=1)


def _vq_body(x_ref, cb_ref, soft_ref, idx_ref, xn_ref):
    x = x_ref[...]
    xn = x / jnp.maximum(jnp.sqrt(jnp.sum(x * x, axis=1, keepdims=True)), 1e-12)
    xn_ref[...] = xn
    cbn = cb_ref[...]
    logits = jax.lax.dot_general(xn, cbn, (((1,), (1,)), ((), ())),
                                 preferred_element_type=jnp.float32)
    d = 2.0 - 2.0 * logits
    e = jnp.exp2(d * _NEGS)
    r = 1.0 / jnp.sum(e, axis=1, keepdims=True)
    soft_ref[...] = e * r
    idx_ref[0, 0, :] = jnp.argmin(d, axis=1).astype(jnp.int32)


def _sc_body(cbn_hbm, idx_hbm, quant_hbm, counts_hbm, idx_v, rows_v, table_v,
             sem):
    c = jax.lax.axis_index("c")
    s = jax.lax.axis_index("s")
    wid = s * _NC + c
    base = wid * _BPW
    pltpu.sync_copy(idx_hbm.at[pl.ds(base, _BPW)], idx_v)
    pltpu.async_copy(cbn_hbm.at[idx_v], rows_v, sem).wait()
    pltpu.sync_copy(rows_v, quant_hbm.at[pl.ds(base, _BPW)])

    def _zero(j, carry):
        table_v[pl.ds(j * 16, 16)] = jnp.zeros((16,), jnp.float32)
        return carry

    jax.lax.fori_loop(0, N_EMB // 16, _zero, 0)

    def _hist(j, carry):
        idx16 = idx_v[pl.ds(j * 16, 16)]
        plsc.addupdate_scatter(table_v, [idx16], jnp.ones((16,), jnp.float32))
        return carry

    jax.lax.fori_loop(0, _BPW // 16, _hist, 0)
    pltpu.sync_copy(table_v, counts_hbm.at[wid])


_sc_gather_hist = functools.partial(
    pl.kernel,
    mesh=plsc.VectorSubcoreMesh(core_axis_name="c", subcore_axis_name="s"),
    out_type=[
        jax.ShapeDtypeStruct((N_TOK, _PAD), jnp.float32),
        jax.ShapeDtypeStruct((_NW, N_EMB), jnp.float32),
    ],
    scratch_types=[
        pltpu.VMEM((_BPW,), jnp.int32),
        pltpu.VMEM((_BPW, _PAD), jnp.float32),
        pltpu.VMEM((N_EMB,), jnp.float32),
        pltpu.SemaphoreType.DMA,
    ],
    compiler_params=pltpu.CompilerParams(needs_layout_passes=False),
)(_sc_body)


def _fin_body(cnt_ref, quant_ref, xn_ref, perp_ref, loss_ref):
    counts = jnp.sum(cnt_ref[...], axis=0, keepdims=True)
    avg = counts / N_TOK
    perp_ref[...] = jnp.reshape(
        jnp.exp(-jnp.sum(avg * jnp.log(avg + 1e-10))), (1, 1))
    dq = quant_ref[...][:, :DIM] - xn_ref[...]
    loss_ref[...] = jnp.reshape(
        0.25 * jnp.sum(dq * dq) / (N_TOK * DIM), (1, 1))


def kernel(inputs, codebook):
    flat = inputs.reshape(-1, DIM)
    cbn, cbnp = pl.pallas_call(
        _prep_body,
        out_shape=[
            jax.ShapeDtypeStruct((N_EMB, DIM), jnp.float32),
            jax.ShapeDtypeStruct((N_EMB, _PAD), jnp.float32),
        ],
    )(codebook)
    soft, idx, xn = pl.pallas_call(
        _vq_body,
        grid=(GRID,),
        in_specs=[
            pl.BlockSpec((TILE, DIM), lambda i: (i, 0)),
            pl.BlockSpec((N_EMB, DIM), lambda i: (0, 0)),
        ],
        out_specs=[
            pl.BlockSpec((TILE, N_EMB), lambda i: (i, 0)),
            pl.BlockSpec((1, 1, TILE), lambda i: (i, 0, 0)),
            pl.BlockSpec((TILE, DIM), lambda i: (i, 0)),
        ],
        out_shape=[
            jax.ShapeDtypeStruct((N_TOK, N_EMB), jnp.float32),
            jax.ShapeDtypeStruct((GRID, 1, TILE), jnp.int32),
            jax.ShapeDtypeStruct((N_TOK, DIM), jnp.float32),
        ],
    )(flat, cbn)
    idx_flat = idx.reshape(-1)
    quant_pad, counts = _sc_gather_hist(cbnp, idx_flat)
    quant = quant_pad[:, :DIM]
    perp, loss = pl.pallas_call(
        _fin_body,
        out_shape=[
            jax.ShapeDtypeStruct((1, 1), jnp.float32),
            jax.ShapeDtypeStruct((1, 1), jnp.float32),
        ],
    )(counts, quant_pad, xn)
    return (loss[0, 0], quant.reshape(inputs.shape), soft, perp[0, 0],
            idx_flat[:, None])


# speedup vs baseline: 1.3478x; 1.1018x over previous
import functools

import jax
import jax.numpy as jnp
from jax.experimental import pallas as pl
from jax.experimental.pallas import tpu as pltpu
from jax.experimental.pallas import tpu_sc as plsc

N_EMB = 8192
DIM = 32
N_TOK = 8192
TILE = 512
GRID = N_TOK // TILE

_NC = 2
_NS = 16
_NW = _NC * _NS
_BPW = N_TOK // _NW


_PAD = 128

_ES = 20.0 * 1.4426950408889634


def _prep_body(cb_ref, cbn_ref, cbnp_ref):
    cb = cb_ref[...]
    n = jnp.sqrt(jnp.sum(cb * cb, axis=1, keepdims=True))
    cbn = cb / jnp.maximum(n, 1e-12)
    cbn_ref[...] = cbn
    cbnp_ref[...] = jnp.concatenate(
        [cbn, jnp.zeros((N_EMB, _PAD - DIM), jnp.float32)], axis=1)


def _vq_body(x_ref, cb_ref, soft_ref, idx_ref, xn_ref):
    x = x_ref[...]
    xn = x / jnp.maximum(jnp.sqrt(jnp.sum(x * x, axis=1, keepdims=True)), 1e-12)
    xn_ref[...] = xn
    cbn = cb_ref[...]
    logits = jax.lax.dot_general(xn, cbn, (((1,), (1,)), ((), ())),
                                 preferred_element_type=jnp.float32)
    e = jnp.exp2(logits * _ES)
    r = 1.0 / jnp.sum(e, axis=1, keepdims=True)
    soft_ref[...] = e * r
    idx_ref[0, 0, :] = jnp.argmax(logits, axis=1).astype(jnp.int32)


def _sc_body(cbn_hbm, idx_hbm, quant_hbm, counts_hbm, idx_v, rows_v, table_v,
             sem):
    c = jax.lax.axis_index("c")
    s = jax.lax.axis_index("s")
    wid = s * _NC + c
    base = wid * _BPW
    pltpu.sync_copy(idx_hbm.at[pl.ds(base, _BPW)], idx_v)
    pltpu.async_copy(cbn_hbm.at[idx_v], rows_v, sem).wait()
    pltpu.sync_copy(rows_v, quant_hbm.at[pl.ds(base, _BPW)])

    def _zero(j, carry):
        table_v[pl.ds(j * 16, 16)] = jnp.zeros((16,), jnp.float32)
        return carry

    jax.lax.fori_loop(0, N_EMB // 16, _zero, 0)

    def _hist(j, carry):
        idx16 = idx_v[pl.ds(j * 16, 16)]
        plsc.addupdate_scatter(table_v, [idx16], jnp.ones((16,), jnp.float32))
        return carry

    jax.lax.fori_loop(0, _BPW // 16, _hist, 0)
    pltpu.sync_copy(table_v, counts_hbm.at[wid])


_sc_gather_hist = functools.partial(
    pl.kernel,
    mesh=plsc.VectorSubcoreMesh(core_axis_name="c", subcore_axis_name="s"),
    out_type=[
        jax.ShapeDtypeStruct((N_TOK, _PAD), jnp.float32),
        jax.ShapeDtypeStruct((_NW, N_EMB), jnp.float32),
    ],
    scratch_types=[
        pltpu.VMEM((_BPW,), jnp.int32),
        pltpu.VMEM((_BPW, _PAD), jnp.float32),
        pltpu.VMEM((N_EMB,), jnp.float32),
        pltpu.SemaphoreType.DMA,
    ],
    compiler_params=pltpu.CompilerParams(needs_layout_passes=False),
)(_sc_body)


def _fin_body(cnt_ref, quant_ref, xn_ref, perp_ref, loss_ref):
    counts = jnp.sum(cnt_ref[...], axis=0, keepdims=True)
    avg = counts / N_TOK
    perp_ref[...] = jnp.reshape(
        jnp.exp(-jnp.sum(avg * jnp.log(avg + 1e-10))), (1, 1))
    dq = quant_ref[...][:, :DIM] - xn_ref[...]
    loss_ref[...] = jnp.reshape(
        0.25 * jnp.sum(dq * dq) / (N_TOK * DIM), (1, 1))


def kernel(inputs, codebook):
    flat = inputs.reshape(-1, DIM)
    cbn, cbnp = pl.pallas_call(
        _prep_body,
        out_shape=[
            jax.ShapeDtypeStruct((N_EMB, DIM), jnp.float32),
            jax.ShapeDtypeStruct((N_EMB, _PAD), jnp.float32),
        ],
    )(codebook)
    soft, idx, xn = pl.pallas_call(
        _vq_body,
        grid=(GRID,),
        in_specs=[
            pl.BlockSpec((TILE, DIM), lambda i: (i, 0)),
            pl.BlockSpec((N_EMB, DIM), lambda i: (0, 0)),
        ],
        out_specs=[
            pl.BlockSpec((TILE, N_EMB), lambda i: (i, 0)),
            pl.BlockSpec((1, 1, TILE), lambda i: (i, 0, 0)),
            pl.BlockSpec((TILE, DIM), lambda i: (i, 0)),
        ],
        out_shape=[
            jax.ShapeDtypeStruct((N_TOK, N_EMB), jnp.float32),
            jax.ShapeDtypeStruct((GRID, 1, TILE), jnp.int32),
            jax.ShapeDtypeStruct((N_TOK, DIM), jnp.float32),
        ],
    )(flat, cbn)
    idx_flat = idx.reshape(-1)
    quant_pad, counts = _sc_gather_hist(cbnp, idx_flat)
    quant = quant_pad[:, :DIM]
    perp, loss = pl.pallas_call(
        _fin_body,
        out_shape=[
            jax.ShapeDtypeStruct((1, 1), jnp.float32),
            jax.ShapeDtypeStruct((1, 1), jnp.float32),
        ],
    )(counts, quant_pad, xn)
    return (loss[0, 0], quant.reshape(inputs.shape), soft, perp[0, 0],
            idx_flat[:, None])
